# baseline (speedup 1.0000x reference)
import functools

import jax
import jax.numpy as jnp
from jax import lax
from jax.experimental import pallas as pl
from jax.experimental.pallas import tpu as pltpu

DIM = 1024
TILE_N = 2048


def _gather_body(idx_ref, row_ref, out_ref):
    out_ref[...] = row_ref[...]


def _matmul_body(h_ref, w_ref, b_ref, out_ref):
    acc = lax.dot_general(
        h_ref[...], w_ref[...],
        dimension_numbers=(((1,), (1,)), ((), ())),
        preferred_element_type=jnp.float32,
    )
    out_ref[...] = acc + b_ref[...]


@jax.jit
def kernel(x, emb_table, W, b):
    B, L = x.shape
    T = B * L
    V, D = W.shape
    idx = x.reshape(T).astype(jnp.int32)

    h = pl.pallas_call(
        _gather_body,
        grid_spec=pltpu.PrefetchScalarGridSpec(
            num_scalar_prefetch=1,
            grid=(T,),
            in_specs=[
                pl.BlockSpec((1, 1, D), lambda i, idx_ref: (idx_ref[i], 0, 0)),
            ],
            out_specs=pl.BlockSpec((1, 1, D), lambda i, idx_ref: (i, 0, 0)),
        ),
        out_shape=jax.ShapeDtypeStruct((T, 1, D), jnp.float32),
    )(idx, emb_table.reshape(V, 1, D))
    h = h.reshape(T, D)

    b2 = b.reshape(1, V)
    n_tiles = pl.cdiv(V, TILE_N)
    logits = pl.pallas_call(
        _matmul_body,
        grid=(n_tiles,),
        in_specs=[
            pl.BlockSpec((T, D), lambda i: (0, 0)),
            pl.BlockSpec((TILE_N, D), lambda i: (i, 0)),
            pl.BlockSpec((1, TILE_N), lambda i: (0, i)),
        ],
        out_specs=pl.BlockSpec((T, TILE_N), lambda i: (0, i)),
        out_shape=jax.ShapeDtypeStruct((T, V), jnp.float32),
        compiler_params=pltpu.CompilerParams(
            dimension_semantics=("arbitrary",),
        ),
    )(h, W, b2)

    return logits.reshape(B, L, V)


# speedup vs baseline: 3.5711x; 3.5711x over previous
import jax
import jax.numpy as jnp
from jax import lax
from jax.experimental import pallas as pl
from jax.experimental.pallas import tpu as pltpu

DIM = 1024
TILE_N = 2048
NUM_TOKENS = 256


def _fused_body(idx_ref, emb_hbm, w_ref, b_ref, out_ref, h_ref, sem):
    i = pl.program_id(0)

    @pl.when(i == 0)
    def _gather():
        def issue(t, _):
            pltpu.make_async_copy(
                emb_hbm.at[pl.ds(idx_ref[t], 1), :],
                h_ref.at[pl.ds(t, 1), :],
                sem,
            ).start()
            return 0

        def wait(t, _):
            pltpu.make_async_copy(
                emb_hbm.at[pl.ds(idx_ref[t], 1), :],
                h_ref.at[pl.ds(t, 1), :],
                sem,
            ).wait()
            return 0

        lax.fori_loop(0, NUM_TOKENS, issue, 0)
        lax.fori_loop(0, NUM_TOKENS, wait, 0)

    acc = lax.dot_general(
        h_ref[...], w_ref[...],
        dimension_numbers=(((1,), (1,)), ((), ())),
        preferred_element_type=jnp.float32,
    )
    out_ref[...] = acc + b_ref[...]


@jax.jit
def kernel(x, emb_table, W, b):
    B, L = x.shape
    T = B * L
    V, D = W.shape
    idx = x.reshape(T).astype(jnp.int32)

    b2 = b.reshape(1, V)
    n_tiles = pl.cdiv(V, TILE_N)
    logits = pl.pallas_call(
        _fused_body,
        grid_spec=pltpu.PrefetchScalarGridSpec(
            num_scalar_prefetch=1,
            grid=(n_tiles,),
            in_specs=[
                pl.BlockSpec(memory_space=pl.ANY),
                pl.BlockSpec((TILE_N, D), lambda i, idx_ref: (i, 0)),
                pl.BlockSpec((1, TILE_N), lambda i, idx_ref: (0, i)),
            ],
            out_specs=pl.BlockSpec((T, TILE_N), lambda i, idx_ref: (0, i)),
            scratch_shapes=[
                pltpu.VMEM((T, D), jnp.float32),
                pltpu.SemaphoreType.DMA,
            ],
        ),
        out_shape=jax.ShapeDtypeStruct((T, V), jnp.float32),
        compiler_params=pltpu.CompilerParams(
            dimension_semantics=("arbitrary",),
        ),
    )(idx, emb_table, W, b2)

    return logits.reshape(B, L, V)


# speedup vs baseline: 3.5773x; 1.0017x over previous
import jax
import jax.numpy as jnp
from jax import lax
from jax.experimental import pallas as pl
from jax.experimental.pallas import tpu as pltpu

DIM = 1024
TILE_N = 4096
NUM_TOKENS = 256


def _fused_body(idx_ref, emb_hbm, w_ref, b_ref, out_ref, h_ref, sem):
    i = pl.program_id(0)

    @pl.when(i == 0)
    def _gather():
        def issue(t, _):
            pltpu.make_async_copy(
                emb_hbm.at[pl.ds(idx_ref[t], 1), :],
                h_ref.at[pl.ds(t, 1), :],
                sem,
            ).start()
            return 0

        def wait(t, _):
            pltpu.make_async_copy(
                emb_hbm.at[pl.ds(idx_ref[t], 1), :],
                h_ref.at[pl.ds(t, 1), :],
                sem,
            ).wait()
            return 0

        lax.fori_loop(0, NUM_TOKENS, issue, 0)
        lax.fori_loop(0, NUM_TOKENS, wait, 0)

    acc = lax.dot_general(
        h_ref[...], w_ref[...],
        dimension_numbers=(((1,), (1,)), ((), ())),
        preferred_element_type=jnp.float32,
    )
    out_ref[...] = acc + b_ref[...]


@jax.jit
def kernel(x, emb_table, W, b):
    B, L = x.shape
    T = B * L
    V, D = W.shape
    idx = x.reshape(T).astype(jnp.int32)

    b2 = b.reshape(1, V)
    n_tiles = pl.cdiv(V, TILE_N)
    logits = pl.pallas_call(
        _fused_body,
        grid_spec=pltpu.PrefetchScalarGridSpec(
            num_scalar_prefetch=1,
            grid=(n_tiles,),
            in_specs=[
                pl.BlockSpec(memory_space=pl.ANY),
                pl.BlockSpec((TILE_N, D), lambda i, idx_ref: (i, 0)),
                pl.BlockSpec((1, TILE_N), lambda i, idx_ref: (0, i)),
            ],
            out_specs=pl.BlockSpec((T, TILE_N), lambda i, idx_ref: (0, i)),
            scratch_shapes=[
                pltpu.VMEM((T, D), jnp.float32),
                pltpu.SemaphoreType.DMA,
            ],
        ),
        out_shape=jax.ShapeDtypeStruct((T, V), jnp.float32),
        compiler_params=pltpu.CompilerParams(
            dimension_semantics=("arbitrary",),
        ),
    )(idx, emb_table, W, b2)

    return logits.reshape(B, L, V)
